# baseline (speedup 1.0000x reference)
import functools

import jax
import jax.numpy as jnp
from jax import lax
from jax.experimental import pallas as pl
from jax.experimental.pallas import tpu as pltpu
from jax.experimental.pallas import tpu_sc as plsc

D = 32
NL = 8


def _gather_sc(xT, W, B, L):
    info = plsc.get_sparse_core_info()
    nw = info.num_cores * info.num_subcores
    assert B == nw * 128
    n_blocks = L // NL

    mesh = plsc.VectorSubcoreMesh(core_axis_name="c", subcore_axis_name="s")

    @functools.partial(
        pl.kernel,
        mesh=mesh,
        out_type=jax.ShapeDtypeStruct((L, D // 8, nw, 8, 128), jnp.float32),
        scratch_types=[
            pltpu.VMEM((2, NL, 128), jnp.int32),
            pltpu.VMEM((2, NL, 128, D), jnp.float32),
            pltpu.VMEM((NL, D // 8, 8, 128), jnp.float32),
            pltpu.SemaphoreType.DMA((2,)),
            pltpu.SemaphoreType.DMA,
        ],
        compiler_params=pltpu.CompilerParams(
            use_tc_tiling_on_sc=False, needs_layout_passes=False),
    )
    def body(x_hbm, w_hbm, out_hbm, idx_v, rows_v, tout_v, gsem, osem):
        wid = lax.axis_index("s") * info.num_cores + lax.axis_index("c")
        b0 = wid * 128

        lanes = [lax.iota(jnp.int32, 16) + 16 * g for g in range(8)]

        def fire_gathers(blk, slot):
            pltpu.sync_copy(x_hbm.at[pl.ds(blk * NL, NL), pl.ds(b0, 128)],
                            idx_v.at[slot])
            for j in range(NL):
                pltpu.async_copy(w_hbm.at[idx_v.at[slot, j]],
                                 rows_v.at[slot, j], gsem.at[slot])

        def store_waits(l0):
            return [
                pltpu.make_async_copy(tout_v.at[j, td],
                                      out_hbm.at[l0 + j, td, wid], osem)
                for j in range(NL) for td in range(D // 8)
            ]

        fire_gathers(0, 0)

        def block(blk, carry):
            p = lax.rem(blk, 2)

            @pl.when(blk != n_blocks - 1)
            def _():
                fire_gathers(blk + 1, 1 - p)

            for j in range(NL):
                pltpu.make_async_copy(w_hbm.at[idx_v.at[p, j]],
                                      rows_v.at[p, j], gsem.at[p]).wait()

            @pl.when(blk != 0)
            def _():
                for c in store_waits((blk - 1) * NL):
                    c.wait()

            l0 = blk * NL
            for j in range(NL):
                src = rows_v.at[p, j]
                for d in range(D):
                    dsplat = jnp.full((16,), d, jnp.int32)
                    for g in range(8):
                        v = plsc.load_gather(src, [lanes[g], dsplat])
                        tout_v[j, d // 8, d % 8, pl.ds(16 * g, 16)] = v
                for td in range(D // 8):
                    pltpu.async_copy(tout_v.at[j, td],
                                     out_hbm.at[l0 + j, td, wid], osem)
            return carry

        lax.fori_loop(0, n_blocks, block, 0)

        for c in store_waits((n_blocks - 1) * NL):
            c.wait()

    return body(xT, W)


def kernel(x, W):
    B, L = x.shape
    xT = jnp.transpose(x.astype(jnp.int32))
    out5 = _gather_sc(xT, W, B, L)
    return out5.transpose(2, 4, 0, 1, 3).reshape(B, L, D)


# speedup vs baseline: 1.4928x; 1.4928x over previous
import functools

import jax
import jax.numpy as jnp
from jax import lax
from jax.experimental import pallas as pl
from jax.experimental.pallas import tpu as pltpu
from jax.experimental.pallas import tpu_sc as plsc

D = 32
NL = 8


def _gather_sc(xT, W, B, L):
    info = plsc.get_sparse_core_info()
    nw = info.num_cores * info.num_subcores
    assert B == nw * 128
    n_blocks = L // NL

    mesh = plsc.VectorSubcoreMesh(core_axis_name="c", subcore_axis_name="s")

    @functools.partial(
        pl.kernel,
        mesh=mesh,
        out_type=jax.ShapeDtypeStruct((L, D // 8, nw, 8, 128), jnp.float32),
        scratch_types=[
            pltpu.VMEM((2, NL, 128), jnp.int32),
            pltpu.VMEM((2, NL, 128, D), jnp.float32),
            pltpu.VMEM((NL, D, 129), jnp.float32),
            pltpu.SemaphoreType.DMA((2,)),
            pltpu.SemaphoreType.DMA,
        ],
        compiler_params=pltpu.CompilerParams(
            use_tc_tiling_on_sc=False, needs_layout_passes=False),
    )
    def body(x_hbm, w_hbm, out_hbm, idx_v, rows_v, tout_v, gsem, osem):
        wid = lax.axis_index("s") * info.num_cores + lax.axis_index("c")
        b0 = wid * 128

        halves = [lax.iota(jnp.int32, 16), lax.iota(jnp.int32, 16) + 16]

        def fire_gathers(blk, slot):
            pltpu.sync_copy(x_hbm.at[pl.ds(blk * NL, NL), pl.ds(b0, 128)],
                            idx_v.at[slot])
            for j in range(NL):
                pltpu.async_copy(w_hbm.at[idx_v.at[slot, j]],
                                 rows_v.at[slot, j], gsem.at[slot])

        def store_copies(l0):
            return [
                pltpu.make_async_copy(
                    tout_v.at[j, pl.ds(8 * td, 8), pl.ds(0, 128)],
                    out_hbm.at[l0 + j, td, wid], osem)
                for j in range(NL) for td in range(D // 8)
            ]

        fire_gathers(0, 0)

        def block(blk, carry):
            p = lax.rem(blk, 2)

            @pl.when(blk != n_blocks - 1)
            def _():
                fire_gathers(blk + 1, 1 - p)

            for j in range(NL):
                pltpu.make_async_copy(w_hbm.at[idx_v.at[p, j]],
                                      rows_v.at[p, j], gsem.at[p]).wait()

            @pl.when(blk != 0)
            def _():
                for c in store_copies((blk - 1) * NL):
                    c.wait()

            l0 = blk * NL
            for j in range(NL):
                src = rows_v.at[p, j]
                dst = tout_v.at[j]
                for i in range(128):
                    isplat = jnp.full((16,), i, jnp.int32)
                    for h in range(2):
                        v = src[i, pl.ds(16 * h, 16)]
                        plsc.store_scatter(dst, [halves[h], isplat], v)
            for c in store_copies(l0):
                c.start()
            return carry

        lax.fori_loop(0, n_blocks, block, 0)

        for c in store_copies((n_blocks - 1) * NL):
            c.wait()

    return body(xT, W)


def kernel(x, W):
    B, L = x.shape
    xT = jnp.transpose(x.astype(jnp.int32))
    out5 = _gather_sc(xT, W, B, L)
    return out5.transpose(2, 4, 0, 1, 3).reshape(B, L, D)


# speedup vs baseline: 2.1335x; 1.4292x over previous
import functools

import jax
import jax.numpy as jnp
from jax import lax
from jax.experimental import pallas as pl
from jax.experimental.pallas import tpu as pltpu
from jax.experimental.pallas import tpu_sc as plsc

D = 32
NL = 8


def _gather_sc(xT, W, B, L):
    info = plsc.get_sparse_core_info()
    nw = info.num_cores * info.num_subcores
    assert B == nw * 128
    n_blocks = L // NL

    mesh = plsc.VectorSubcoreMesh(core_axis_name="c", subcore_axis_name="s")

    @functools.partial(
        pl.kernel,
        mesh=mesh,
        out_type=jax.ShapeDtypeStruct((L, D // 8, nw, 8, 128), jnp.float32),
        scratch_types=[
            pltpu.VMEM((2, NL, 128), jnp.int32),
            pltpu.VMEM((2, NL, 128, D), jnp.float32),
            pltpu.VMEM((NL, D, 129), jnp.float32),
            pltpu.SemaphoreType.DMA((2,)),
            pltpu.SemaphoreType.DMA,
        ],
        compiler_params=pltpu.CompilerParams(
            use_tc_tiling_on_sc=False, needs_layout_passes=False),
    )
    def body(x_hbm, w_hbm, out_hbm, idx_v, rows_v, tout_v, gsem, osem):
        wid = lax.axis_index("s") * info.num_cores + lax.axis_index("c")
        b0 = wid * 128

        halves = [lax.iota(jnp.int32, 16), lax.iota(jnp.int32, 16) + 16]

        def fire_gathers(blk, slot):
            pltpu.sync_copy(x_hbm.at[pl.ds(blk * NL, NL), pl.ds(b0, 128)],
                            idx_v.at[slot])
            for j in range(NL):
                pltpu.async_copy(w_hbm.at[idx_v.at[slot, j]],
                                 rows_v.at[slot, j], gsem.at[slot])

        def store_copies(l0):
            return [
                pltpu.make_async_copy(
                    tout_v.at[j, pl.ds(8 * td, 8), pl.ds(0, 128)],
                    out_hbm.at[l0 + j, td, wid], osem)
                for j in range(NL) for td in range(D // 8)
            ]

        fire_gathers(0, 0)

        def block(blk, carry):
            p = lax.rem(blk, 2)

            @pl.when(blk != n_blocks - 1)
            def _():
                fire_gathers(blk + 1, 1 - p)

            for j in range(NL):
                pltpu.make_async_copy(w_hbm.at[idx_v.at[p, j]],
                                      rows_v.at[p, j], gsem.at[p]).wait()

            @pl.when(blk != 0)
            def _():
                for c in store_copies((blk - 1) * NL):
                    c.wait()

            l0 = blk * NL
            for j in range(NL):
                src = rows_v.at[p, j]
                dst = tout_v.at[j]

                @plsc.parallel_loop(0, 128, unroll=8)
                def _(i):
                    isplat = jnp.full((16,), i, jnp.int32)
                    for h in range(2):
                        v = src[i, pl.ds(16 * h, 16)]
                        plsc.store_scatter(dst, [halves[h], isplat], v)
            for c in store_copies(l0):
                c.start()
            return carry

        lax.fori_loop(0, n_blocks, block, 0)

        for c in store_copies((n_blocks - 1) * NL):
            c.wait()

    return body(xT, W)


def kernel(x, W):
    B, L = x.shape
    xT = jnp.transpose(x.astype(jnp.int32))
    out5 = _gather_sc(xT, W, B, L)
    return out5.transpose(2, 4, 0, 1, 3).reshape(B, L, D)


# speedup vs baseline: 2.1733x; 1.0186x over previous
import functools

import jax
import jax.numpy as jnp
from jax import lax
from jax.experimental import pallas as pl
from jax.experimental.pallas import tpu as pltpu
from jax.experimental.pallas import tpu_sc as plsc

D = 32
NL = 8


def _gather_sc(xT, W, B, L):
    info = plsc.get_sparse_core_info()
    nw = info.num_cores * info.num_subcores
    assert B == nw * 128
    n_blocks = L // NL

    mesh = plsc.VectorSubcoreMesh(core_axis_name="c", subcore_axis_name="s")

    @functools.partial(
        pl.kernel,
        mesh=mesh,
        out_type=jax.ShapeDtypeStruct((L, D // 8, nw, 8, 128), jnp.float32),
        scratch_types=[
            pltpu.VMEM((2, NL, 128), jnp.int32),
            pltpu.VMEM((2, NL, 128, D), jnp.float32),
            pltpu.VMEM((NL, D, 129), jnp.float32),
            pltpu.SemaphoreType.DMA((2,)),
            pltpu.SemaphoreType.DMA,
        ],
        compiler_params=pltpu.CompilerParams(
            use_tc_tiling_on_sc=False, needs_layout_passes=False),
    )
    def body(x_hbm, w_hbm, out_hbm, idx_v, rows_v, tout_v, gsem, osem):
        wid = lax.axis_index("s") * info.num_cores + lax.axis_index("c")
        b0 = wid * 128

        halves = [lax.iota(jnp.int32, 16), lax.iota(jnp.int32, 16) + 16]

        def fire_gathers(blk, slot):
            pltpu.sync_copy(x_hbm.at[pl.ds(blk * NL, NL), pl.ds(b0, 128)],
                            idx_v.at[slot])
            for j in range(NL):
                pltpu.async_copy(w_hbm.at[idx_v.at[slot, j]],
                                 rows_v.at[slot, j], gsem.at[slot])

        def store_copies(l0):
            return [
                pltpu.make_async_copy(
                    tout_v.at[j, pl.ds(8 * td, 8), pl.ds(0, 128)],
                    out_hbm.at[l0 + j, td, wid], osem)
                for j in range(NL) for td in range(D // 8)
            ]

        fire_gathers(0, 0)

        def block(blk, carry):
            p = lax.rem(blk, 2)

            @pl.when(blk != n_blocks - 1)
            def _():
                fire_gathers(blk + 1, 1 - p)

            for j in range(NL):
                pltpu.make_async_copy(w_hbm.at[idx_v.at[p, j]],
                                      rows_v.at[p, j], gsem.at[p]).wait()

            @pl.when(blk != 0)
            def _():
                for c in store_copies((blk - 1) * NL):
                    c.wait()

            l0 = blk * NL
            for j in range(NL):
                src = rows_v.at[p, j]
                dst = tout_v.at[j]

                @plsc.parallel_loop(0, 128, unroll=8)
                def _(i):
                    isplat = jnp.full((16,), i, jnp.int32)
                    for h in range(2):
                        v = src[i, pl.ds(16 * h, 16)]
                        plsc.store_scatter(dst, [halves[h], isplat], v)
            for c in store_copies(l0):
                c.start()
            return carry

        lax.fori_loop(0, n_blocks, block, 0)

        for c in store_copies((n_blocks - 1) * NL):
            c.wait()

    return body(xT, W)


def kernel(x, W):
    B, L = x.shape
    xT = jnp.transpose(x.astype(jnp.int32) * 4)
    W4 = jnp.pad(W, ((0, 0), (0, 96))).reshape(4 * W.shape[0], D)
    out5 = _gather_sc(xT, W4, B, L)
    return out5.transpose(2, 4, 0, 1, 3).reshape(B, L, D)
